# baseline (speedup 1.0000x reference)
import jax
import jax.numpy as jnp
from jax.experimental import pallas as pl
from jax.experimental.pallas import tpu as pltpu

_NUM_CODES = 8192
_EMBED_DIM = 32
_TILE = 256
_CHUNK = 2048


def _vq_kernel(x_ref, e_ref, qo_ref, q_ref, idx_ref, acc_ref, e2_ref):
    i = pl.program_id(0)
    n_steps = pl.num_programs(0)
    x = x_ref[...]
    e = e_ref[...]

    @pl.when(i == 0)
    def _():
        ones = jnp.ones((1, _EMBED_DIM), dtype=jnp.float32)
        e2_ref[...] = jax.lax.dot_general(
            ones, e * e,
            dimension_numbers=(((1,), (1,)), ((), ())),
            precision=jax.lax.Precision.HIGHEST,
            preferred_element_type=jnp.float32)
        acc_ref[...] = jnp.zeros((1, 1), jnp.float32)

    xx = x * x
    r = xx[:, 0:8]
    for j in range(1, 4):
        r = r + xx[:, 8 * j:8 * j + 8]
    s = r[:, 0:4] + r[:, 4:8]
    s = s[:, 0:2] + s[:, 2:4]
    x2 = s[:, 0:1] + s[:, 1:2]
    xb = x.astype(jnp.bfloat16)
    eb = e.astype(jnp.bfloat16)
    mm = jax.lax.dot_general(
        xb, eb,
        dimension_numbers=(((1,), (1,)), ((), ())),
        preferred_element_type=jnp.float32)
    d = (x2 + e2_ref[...]) - 2.0 * mm

    acc_v = None
    acc_i = None
    for c in range(_NUM_CODES // _CHUNK):
        dc = d[:, c * _CHUNK:(c + 1) * _CHUNK]
        md = jnp.min(dc, axis=1, keepdims=True)
        iota_c = jax.lax.broadcasted_iota(jnp.int32, (_TILE, _CHUNK), 1)
        ic = jnp.min(jnp.where(dc == md, iota_c + c * _CHUNK, _NUM_CODES),
                     axis=1, keepdims=True)
        md_b = md.astype(jnp.bfloat16).astype(jnp.float32)
        if acc_v is None:
            acc_v, acc_i = md_b, ic
        else:
            upd = md < acc_v
            acc_v = jnp.where(upd, md_b, acc_v)
            acc_i = jnp.where(upd, ic, acc_i)
    idx = acc_i

    iota = jax.lax.broadcasted_iota(jnp.int32, (_TILE, _NUM_CODES), 1)
    onehot = (iota == idx).astype(jnp.float32)
    q = jax.lax.dot_general(
        onehot, e,
        dimension_numbers=(((1,), (0,)), ((), ())),
        precision=jax.lax.Precision.HIGHEST,
        preferred_element_type=jnp.float32)

    qo_ref[...] = x + (q - x)
    q_ref[...] = q
    idx_ref[...] = idx[None]

    dq = q - x
    acc_ref[...] += jnp.sum(dq * dq, axis=(0, 1), keepdims=True)

    @pl.when(i == n_steps - 1)
    def _():
        acc_ref[...] = acc_ref[...] / (
            jnp.float32(n_steps) * _TILE * _EMBED_DIM)


@jax.jit
def kernel(inputs, embedding):
    n_tok = inputs.shape[0] * inputs.shape[1]
    flat = inputs.reshape(n_tok, _EMBED_DIM)
    grid = n_tok // _TILE

    qo, q, idx3, loss2 = pl.pallas_call(
        _vq_kernel,
        grid=(grid,),
        in_specs=[
            pl.BlockSpec((_TILE, _EMBED_DIM), lambda i: (i, 0)),
            pl.BlockSpec((_NUM_CODES, _EMBED_DIM), lambda i: (0, 0)),
        ],
        out_specs=[
            pl.BlockSpec((_TILE, _EMBED_DIM), lambda i: (i, 0)),
            pl.BlockSpec((_TILE, _EMBED_DIM), lambda i: (i, 0)),
            pl.BlockSpec((1, _TILE, 1), lambda i: (i, 0, 0)),
            pl.BlockSpec((1, 1), lambda i: (0, 0)),
        ],
        out_shape=[
            jax.ShapeDtypeStruct((n_tok, _EMBED_DIM), jnp.float32),
            jax.ShapeDtypeStruct((n_tok, _EMBED_DIM), jnp.float32),
            jax.ShapeDtypeStruct((grid, _TILE, 1), jnp.int32),
            jax.ShapeDtypeStruct((1, 1), jnp.float32),
        ],
        scratch_shapes=[pltpu.VMEM((1, _NUM_CODES), jnp.float32)],
    )(flat, embedding)

    loss = loss2[0, 0]
    return (qo.reshape(inputs.shape), q.reshape(inputs.shape),
            loss, loss, idx3.reshape(inputs.shape[:-1]))


# speedup vs baseline: 2.0314x; 2.0314x over previous
import functools

import jax
import jax.numpy as jnp
from jax import lax
from jax.experimental import pallas as pl
from jax.experimental.pallas import tpu as pltpu
from jax.experimental.pallas import tpu_sc as plsc

_NUM_CODES = 8192
_EMBED_DIM = 32
_TILE = 256
_CHUNK = 2048
_PACK = 128 // _EMBED_DIM


def _argmin_kernel(x_ref, e_ref, idx_ref, e2_ref):
    i = pl.program_id(0)
    x = x_ref[...]
    e = e_ref[...]

    @pl.when(i == 0)
    def _():
        ones = jnp.ones((1, _EMBED_DIM), dtype=jnp.float32)
        e2_ref[...] = jax.lax.dot_general(
            ones, e * e,
            dimension_numbers=(((1,), (1,)), ((), ())),
            precision=jax.lax.Precision.HIGHEST,
            preferred_element_type=jnp.float32)

    xx = x * x
    r = xx[:, 0:8]
    for j in range(1, 4):
        r = r + xx[:, 8 * j:8 * j + 8]
    s = r[:, 0:4] + r[:, 4:8]
    s = s[:, 0:2] + s[:, 2:4]
    x2 = s[:, 0:1] + s[:, 1:2]

    xb = x.astype(jnp.bfloat16)
    eb = e.astype(jnp.bfloat16)
    mm = jax.lax.dot_general(
        xb, eb,
        dimension_numbers=(((1,), (1,)), ((), ())),
        preferred_element_type=jnp.float32)
    d = (x2 + e2_ref[...]) - 2.0 * mm

    acc_v = None
    acc_i = None
    for c in range(_NUM_CODES // _CHUNK):
        dc = d[:, c * _CHUNK:(c + 1) * _CHUNK]
        md = jnp.min(dc, axis=1, keepdims=True)
        iota_c = jax.lax.broadcasted_iota(jnp.int32, (_TILE, _CHUNK), 1)
        ic = jnp.min(jnp.where(dc == md, iota_c + c * _CHUNK, _NUM_CODES),
                     axis=1, keepdims=True)
        md_b = md.astype(jnp.bfloat16).astype(jnp.float32)
        if acc_v is None:
            acc_v, acc_i = md_b, ic
        else:
            upd = md < acc_v
            acc_v = jnp.where(upd, md_b, acc_v)
            acc_i = jnp.where(upd, ic, acc_i)

    idx_ref[...] = acc_i[None]


def _finish_kernel(x_ref, q128_ref, idx_ref, qo_ref, q_ref, loss_ref):
    i = pl.program_id(0)
    n_steps = pl.num_programs(0)
    x = x_ref[...]
    q128 = q128_ref[...]
    rem = idx_ref[0] & (_PACK - 1)

    q = jnp.where(rem == 0, q128[:, 0:_EMBED_DIM], 0.0)
    for k in range(1, _PACK):
        q = jnp.where(rem == k,
                      q128[:, k * _EMBED_DIM:(k + 1) * _EMBED_DIM], q)

    qo_ref[...] = x + (q - x)
    q_ref[...] = q

    @pl.when(i == 0)
    def _():
        loss_ref[...] = jnp.zeros((1, 1), jnp.float32)

    dq = q - x
    loss_ref[...] += jnp.sum(dq * dq, axis=(0, 1), keepdims=True)

    @pl.when(i == n_steps - 1)
    def _():
        loss_ref[...] = loss_ref[...] / (
            jnp.float32(n_steps) * _TILE * _EMBED_DIM)


def _sc_gather(table128, idx128):
    info = plsc.get_sparse_core_info()
    n_workers = info.num_cores * info.num_subcores
    b_per_w = idx128.shape[0] // n_workers
    mesh = plsc.VectorSubcoreMesh(core_axis_name="c", subcore_axis_name="s")

    @functools.partial(
        pl.kernel, mesh=mesh,
        out_type=jax.ShapeDtypeStruct((idx128.shape[0], 128), jnp.float32),
        scratch_types=[
            pltpu.VMEM((b_per_w,), jnp.int32),
            pltpu.VMEM((b_per_w, 128), jnp.float32),
            pltpu.SemaphoreType.DMA,
        ],
    )
    def gather(table_hbm, idx_hbm, out_hbm, idx_v, rows_v, sem):
        wid = lax.axis_index("s") * info.num_cores + lax.axis_index("c")
        base = wid * b_per_w
        pltpu.sync_copy(idx_hbm.at[pl.ds(base, b_per_w)], idx_v)
        pltpu.async_copy(table_hbm.at[idx_v], rows_v, sem).wait()
        pltpu.sync_copy(rows_v, out_hbm.at[pl.ds(base, b_per_w)])

    return gather(table128, idx128)


@jax.jit
def kernel(inputs, embedding):
    n_tok = inputs.shape[0] * inputs.shape[1]
    flat = inputs.reshape(n_tok, _EMBED_DIM)
    grid = n_tok // _TILE

    idx3 = pl.pallas_call(
        _argmin_kernel,
        grid=(grid,),
        in_specs=[
            pl.BlockSpec((_TILE, _EMBED_DIM), lambda i: (i, 0)),
            pl.BlockSpec((_NUM_CODES, _EMBED_DIM), lambda i: (0, 0)),
        ],
        out_specs=pl.BlockSpec((1, _TILE, 1), lambda i: (i, 0, 0)),
        out_shape=jax.ShapeDtypeStruct((grid, _TILE, 1), jnp.int32),
        scratch_shapes=[pltpu.VMEM((1, _NUM_CODES), jnp.float32)],
    )(flat, embedding)

    idx128 = idx3.reshape(n_tok) >> 2
    table128 = embedding.reshape(_NUM_CODES // _PACK, 128)
    q128 = _sc_gather(table128, idx128)

    qo, q, loss2 = pl.pallas_call(
        _finish_kernel,
        grid=(grid,),
        in_specs=[
            pl.BlockSpec((_TILE, _EMBED_DIM), lambda i: (i, 0)),
            pl.BlockSpec((_TILE, 128), lambda i: (i, 0)),
            pl.BlockSpec((1, _TILE, 1), lambda i: (i, 0, 0)),
        ],
        out_specs=[
            pl.BlockSpec((_TILE, _EMBED_DIM), lambda i: (i, 0)),
            pl.BlockSpec((_TILE, _EMBED_DIM), lambda i: (i, 0)),
            pl.BlockSpec((1, 1), lambda i: (0, 0)),
        ],
        out_shape=[
            jax.ShapeDtypeStruct((n_tok, _EMBED_DIM), jnp.float32),
            jax.ShapeDtypeStruct((n_tok, _EMBED_DIM), jnp.float32),
            jax.ShapeDtypeStruct((1, 1), jnp.float32),
        ],
    )(flat, q128, idx3)

    loss = loss2[0, 0]
    return (qo.reshape(inputs.shape), q.reshape(inputs.shape),
            loss, loss, idx3.reshape(inputs.shape[:-1]))


# speedup vs baseline: 2.3653x; 1.1643x over previous
import functools

import jax
import jax.numpy as jnp
from jax import lax
from jax.experimental import pallas as pl
from jax.experimental.pallas import tpu as pltpu
from jax.experimental.pallas import tpu_sc as plsc

_NUM_CODES = 8192
_EMBED_DIM = 32
_TILE = 256
_CHUNK = 2048
_LANES = 128
_PACK = 128 // _EMBED_DIM


def _argmin_kernel(x_ref, e_ref, idx_ref, e2_ref):
    i = pl.program_id(0)
    x = x_ref[...]
    e = e_ref[...]

    @pl.when(i == 0)
    def _():
        ones = jnp.ones((1, _EMBED_DIM), dtype=jnp.float32)
        e2_ref[...] = jax.lax.dot_general(
            ones, e * e,
            dimension_numbers=(((1,), (1,)), ((), ())),
            precision=jax.lax.Precision.HIGHEST,
            preferred_element_type=jnp.float32)

    xx = x * x
    r = xx[:, 0:8]
    for j in range(1, 4):
        r = r + xx[:, 8 * j:8 * j + 8]
    s = r[:, 0:4] + r[:, 4:8]
    s = s[:, 0:2] + s[:, 2:4]
    x2 = s[:, 0:1] + s[:, 1:2]

    xb = x.astype(jnp.bfloat16)
    eb = e.astype(jnp.bfloat16)
    e2 = e2_ref[...]
    lane = jax.lax.broadcasted_iota(jnp.int32, (_TILE, _LANES), 1)

    acc_v = None
    acc_i = None
    n_blk = _CHUNK // _LANES
    for c in range(_NUM_CODES // _CHUNK):
        mm = jax.lax.dot_general(
            xb, eb[c * _CHUNK:(c + 1) * _CHUNK],
            dimension_numbers=(((1,), (1,)), ((), ())),
            preferred_element_type=jnp.float32)
        run_v = None
        run_b = None
        for b in range(n_blk):
            lo = b * _LANES
            e2b = e2[:, c * _CHUNK + lo:c * _CHUNK + lo + _LANES]
            db = (x2 + e2b) - 2.0 * mm[:, lo:lo + _LANES]
            if run_v is None:
                run_v = db
                run_b = jnp.zeros((_TILE, _LANES), jnp.int32)
            else:
                upd = db < run_v
                run_v = jnp.where(upd, db, run_v)
                run_b = jnp.where(upd, b, run_b)
        md = jnp.min(run_v, axis=1, keepdims=True)
        cand = run_b * _LANES + lane
        ic = jnp.min(jnp.where(run_v == md, cand, _NUM_CODES),
                     axis=1, keepdims=True) + c * _CHUNK
        md_b = md.astype(jnp.bfloat16).astype(jnp.float32)
        if acc_v is None:
            acc_v, acc_i = md_b, ic
        else:
            upd = md < acc_v
            acc_v = jnp.where(upd, md_b, acc_v)
            acc_i = jnp.where(upd, ic, acc_i)

    idx_ref[...] = acc_i[None]


def _finish_kernel(x_ref, q128_ref, idx_ref, qo_ref, q_ref, loss_ref):
    x = x_ref[...]
    q128 = q128_ref[...]
    rem = idx_ref[...] & (_PACK - 1)

    q = jnp.where(rem == 0, q128[:, 0:_EMBED_DIM], 0.0)
    for k in range(1, _PACK):
        q = jnp.where(rem == k,
                      q128[:, k * _EMBED_DIM:(k + 1) * _EMBED_DIM], q)

    qo_ref[...] = x + (q - x)
    q_ref[...] = q
    dq = q - x
    loss_ref[...] = jnp.sum(dq * dq, axis=(0, 1), keepdims=True) / (
        jnp.float32(x.shape[0]) * x.shape[1])


def _sc_gather(table128, idx128):
    info = plsc.get_sparse_core_info()
    n_workers = info.num_cores * info.num_subcores
    b_per_w = idx128.shape[0] // n_workers
    mesh = plsc.VectorSubcoreMesh(core_axis_name="c", subcore_axis_name="s")

    @functools.partial(
        pl.kernel, mesh=mesh,
        out_type=jax.ShapeDtypeStruct((idx128.shape[0], 128), jnp.float32),
        scratch_types=[
            pltpu.VMEM((b_per_w,), jnp.int32),
            pltpu.VMEM((b_per_w, 128), jnp.float32),
            pltpu.SemaphoreType.DMA,
        ],
    )
    def gather(table_hbm, idx_hbm, out_hbm, idx_v, rows_v, sem):
        wid = lax.axis_index("s") * info.num_cores + lax.axis_index("c")
        base = wid * b_per_w
        pltpu.sync_copy(idx_hbm.at[pl.ds(base, b_per_w)], idx_v)
        pltpu.async_copy(table_hbm.at[idx_v], rows_v, sem).wait()
        pltpu.sync_copy(rows_v, out_hbm.at[pl.ds(base, b_per_w)])

    return gather(table128, idx128)


@jax.jit
def kernel(inputs, embedding):
    n_tok = inputs.shape[0] * inputs.shape[1]
    flat = inputs.reshape(n_tok, _EMBED_DIM)
    grid = n_tok // _TILE

    idx3 = pl.pallas_call(
        _argmin_kernel,
        grid=(grid,),
        in_specs=[
            pl.BlockSpec((_TILE, _EMBED_DIM), lambda i: (i, 0)),
            pl.BlockSpec((_NUM_CODES, _EMBED_DIM), lambda i: (0, 0)),
        ],
        out_specs=pl.BlockSpec((1, _TILE, 1), lambda i: (i, 0, 0)),
        out_shape=jax.ShapeDtypeStruct((grid, _TILE, 1), jnp.int32),
        scratch_shapes=[pltpu.VMEM((1, _NUM_CODES), jnp.float32)],
    )(flat, embedding)

    idx128 = idx3.reshape(n_tok) >> 2
    table128 = embedding.reshape(_NUM_CODES // _PACK, 128)
    q128 = _sc_gather(table128, idx128)

    qo, q, loss2 = pl.pallas_call(
        _finish_kernel,
        out_shape=[
            jax.ShapeDtypeStruct((n_tok, _EMBED_DIM), jnp.float32),
            jax.ShapeDtypeStruct((n_tok, _EMBED_DIM), jnp.float32),
            jax.ShapeDtypeStruct((1, 1), jnp.float32),
        ],
    )(flat, q128, idx3.reshape(n_tok, 1))

    loss = loss2[0, 0]
    return (qo.reshape(inputs.shape), q.reshape(inputs.shape),
            loss, loss, idx3.reshape(inputs.shape[:-1]))


# speedup vs baseline: 2.4773x; 1.0474x over previous
import functools

import jax
import jax.numpy as jnp
from jax import lax
from jax.experimental import pallas as pl
from jax.experimental.pallas import tpu as pltpu
from jax.experimental.pallas import tpu_sc as plsc

_NUM_CODES = 8192
_EMBED_DIM = 32
_TILE = 256
_CHUNK = 2048
_LANES = 128
_PACK = 128 // _EMBED_DIM


def _prep_kernel(e_ref, e2_ref, eb_ref):
    e = e_ref[...]
    ones = jnp.ones((1, _EMBED_DIM), dtype=jnp.float32)
    e2_ref[...] = jax.lax.dot_general(
        ones, e * e,
        dimension_numbers=(((1,), (1,)), ((), ())),
        precision=jax.lax.Precision.HIGHEST,
        preferred_element_type=jnp.float32)
    eb_ref[...] = e.astype(jnp.bfloat16)


def _argmin_kernel(x_ref, eb_ref, e2_ref, idx_ref):
    x = x_ref[...]
    eb = eb_ref[...]

    xx = x * x
    r = xx[:, 0:8]
    for j in range(1, 4):
        r = r + xx[:, 8 * j:8 * j + 8]
    s = r[:, 0:4] + r[:, 4:8]
    s = s[:, 0:2] + s[:, 2:4]
    x2 = s[:, 0:1] + s[:, 1:2]

    xb = x.astype(jnp.bfloat16)
    e2 = e2_ref[...]
    lane = jax.lax.broadcasted_iota(jnp.int32, (_TILE, _LANES), 1)

    acc_v = None
    acc_i = None
    n_blk = _CHUNK // _LANES
    for c in range(_NUM_CODES // _CHUNK):
        mm = jax.lax.dot_general(
            xb, eb[c * _CHUNK:(c + 1) * _CHUNK, :],
            dimension_numbers=(((1,), (1,)), ((), ())),
            preferred_element_type=jnp.float32)
        run_v = None
        run_b = None
        for b in range(n_blk):
            lo = b * _LANES
            e2b = e2[:, c * _CHUNK + lo:c * _CHUNK + lo + _LANES]
            db = (x2 + e2b) - 2.0 * mm[:, lo:lo + _LANES]
            if run_v is None:
                run_v = db
                run_b = jnp.zeros((_TILE, _LANES), jnp.int32)
            else:
                upd = db < run_v
                run_v = jnp.where(upd, db, run_v)
                run_b = jnp.where(upd, b, run_b)
        md = jnp.min(run_v, axis=1, keepdims=True)
        cand = run_b * _LANES + lane
        ic = jnp.min(jnp.where(run_v == md, cand, _NUM_CODES),
                     axis=1, keepdims=True) + c * _CHUNK
        md_b = md.astype(jnp.bfloat16).astype(jnp.float32)
        if acc_v is None:
            acc_v, acc_i = md_b, ic
        else:
            upd = md < acc_v
            acc_v = jnp.where(upd, md_b, acc_v)
            acc_i = jnp.where(upd, ic, acc_i)

    idx_ref[...] = acc_i[None]


def _finish_kernel(x_ref, q128_ref, idx_ref, qo_ref, q_ref, loss_ref):
    x = x_ref[...]
    q128 = q128_ref[...]
    rem = idx_ref[...] & (_PACK - 1)

    q = jnp.where(rem == 0, q128[:, 0:_EMBED_DIM], 0.0)
    for k in range(1, _PACK):
        q = jnp.where(rem == k,
                      q128[:, k * _EMBED_DIM:(k + 1) * _EMBED_DIM], q)

    qo_ref[...] = x + (q - x)
    q_ref[...] = q
    dq = q - x
    loss_ref[...] = jnp.sum(dq * dq, axis=(0, 1), keepdims=True) / (
        jnp.float32(x.shape[0]) * x.shape[1])


def _sc_gather(table128, idx128):
    info = plsc.get_sparse_core_info()
    n_workers = info.num_cores * info.num_subcores
    b_per_w = idx128.shape[0] // n_workers
    mesh = plsc.VectorSubcoreMesh(core_axis_name="c", subcore_axis_name="s")

    @functools.partial(
        pl.kernel, mesh=mesh,
        out_type=jax.ShapeDtypeStruct((idx128.shape[0], 128), jnp.float32),
        scratch_types=[
            pltpu.VMEM((b_per_w,), jnp.int32),
            pltpu.VMEM((b_per_w, 128), jnp.float32),
            pltpu.SemaphoreType.DMA,
        ],
    )
    def gather(table_hbm, idx_hbm, out_hbm, idx_v, rows_v, sem):
        wid = lax.axis_index("s") * info.num_cores + lax.axis_index("c")
        base = wid * b_per_w
        pltpu.sync_copy(idx_hbm.at[pl.ds(base, b_per_w)], idx_v)
        pltpu.async_copy(table_hbm.at[idx_v], rows_v, sem).wait()
        pltpu.sync_copy(rows_v, out_hbm.at[pl.ds(base, b_per_w)])

    return gather(table128, idx128)


@jax.jit
def kernel(inputs, embedding):
    n_tok = inputs.shape[0] * inputs.shape[1]
    flat = inputs.reshape(n_tok, _EMBED_DIM)
    grid = n_tok // _TILE

    e2, eb = pl.pallas_call(
        _prep_kernel,
        out_shape=[
            jax.ShapeDtypeStruct((1, _NUM_CODES), jnp.float32),
            jax.ShapeDtypeStruct((_NUM_CODES, _EMBED_DIM), jnp.bfloat16),
        ],
    )(embedding)

    idx3 = pl.pallas_call(
        _argmin_kernel,
        grid=(grid,),
        in_specs=[
            pl.BlockSpec((_TILE, _EMBED_DIM), lambda i: (i, 0)),
            pl.BlockSpec((_NUM_CODES, _EMBED_DIM), lambda i: (0, 0)),
            pl.BlockSpec((1, _NUM_CODES), lambda i: (0, 0)),
        ],
        out_specs=pl.BlockSpec((1, _TILE, 1), lambda i: (i, 0, 0)),
        out_shape=jax.ShapeDtypeStruct((grid, _TILE, 1), jnp.int32),
    )(flat, eb, e2)

    idx128 = idx3.reshape(n_tok) >> 2
    table128 = embedding.reshape(_NUM_CODES // _PACK, 128)
    q128 = _sc_gather(table128, idx128)

    qo, q, loss2 = pl.pallas_call(
        _finish_kernel,
        out_shape=[
            jax.ShapeDtypeStruct((n_tok, _EMBED_DIM), jnp.float32),
            jax.ShapeDtypeStruct((n_tok, _EMBED_DIM), jnp.float32),
            jax.ShapeDtypeStruct((1, 1), jnp.float32),
        ],
    )(flat, q128, idx3.reshape(n_tok, 1))

    loss = loss2[0, 0]
    return (qo.reshape(inputs.shape), q.reshape(inputs.shape),
            loss, loss, idx3.reshape(inputs.shape[:-1]))


# speedup vs baseline: 2.5027x; 1.0102x over previous
import functools

import jax
import jax.numpy as jnp
from jax import lax
from jax.experimental import pallas as pl
from jax.experimental.pallas import tpu as pltpu
from jax.experimental.pallas import tpu_sc as plsc

_NUM_CODES = 8192
_EMBED_DIM = 32
_TILE = 256
_CHUNK = 2048
_LANES = 128
_PACK = 128 // _EMBED_DIM


def _prep_kernel(e_ref, e2_ref, eb_ref):
    e = e_ref[...]
    ones = jnp.ones((1, _EMBED_DIM), dtype=jnp.float32)
    e2_ref[...] = jax.lax.dot_general(
        ones, e * e,
        dimension_numbers=(((1,), (1,)), ((), ())),
        precision=jax.lax.Precision.HIGHEST,
        preferred_element_type=jnp.float32)
    eb_ref[...] = e.astype(jnp.bfloat16)


def _argmin_kernel(x_ref, eb_ref, e2_ref, idx_ref):
    x = x_ref[...]
    eb = eb_ref[...]

    xx = x * x
    r = xx[:, 0:8]
    for j in range(1, 4):
        r = r + xx[:, 8 * j:8 * j + 8]
    s = r[:, 0:4] + r[:, 4:8]
    s = s[:, 0:2] + s[:, 2:4]
    x2 = s[:, 0:1] + s[:, 1:2]

    xb = x.astype(jnp.bfloat16)
    e2 = e2_ref[...]
    lane = jax.lax.broadcasted_iota(jnp.int32, (_TILE, _LANES), 1)

    acc_v = None
    acc_i = None
    n_blk = _CHUNK // _LANES
    for c in range(_NUM_CODES // _CHUNK):
        mm = jax.lax.dot_general(
            xb, eb[c * _CHUNK:(c + 1) * _CHUNK, :],
            dimension_numbers=(((1,), (1,)), ((), ())),
            preferred_element_type=jnp.float32)
        run_v = None
        run_b = None
        for b in range(n_blk):
            lo = b * _LANES
            e2b = e2[:, c * _CHUNK + lo:c * _CHUNK + lo + _LANES]
            db = (x2 + e2b) - 2.0 * mm[:, lo:lo + _LANES]
            if run_v is None:
                run_v = db
                run_b = jnp.zeros((_TILE, _LANES), jnp.int32)
            else:
                upd = db < run_v
                run_v = jnp.where(upd, db, run_v)
                run_b = jnp.where(upd, b, run_b)
        md = jnp.min(run_v, axis=1, keepdims=True)
        cand = run_b * _LANES + lane
        ic = jnp.min(jnp.where(run_v == md, cand, _NUM_CODES),
                     axis=1, keepdims=True) + c * _CHUNK
        md_b = md.astype(jnp.bfloat16).astype(jnp.float32)
        if acc_v is None:
            acc_v, acc_i = md_b, ic
        else:
            upd = md < acc_v
            acc_v = jnp.where(upd, md_b, acc_v)
            acc_i = jnp.where(upd, ic, acc_i)

    idx_ref[...] = acc_i[None]


def _finish_kernel(x_ref, q128_ref, idx_ref, qo_ref, q_ref, loss_ref):
    i = pl.program_id(0)
    n_steps = pl.num_programs(0)
    x = x_ref[...]
    q128 = q128_ref[...]
    rem = idx_ref[...] & (_PACK - 1)

    q = jnp.where(rem == 0, q128[:, 0:_EMBED_DIM], 0.0)
    for k in range(1, _PACK):
        q = jnp.where(rem == k,
                      q128[:, k * _EMBED_DIM:(k + 1) * _EMBED_DIM], q)

    qo_ref[...] = x + (q - x)
    q_ref[...] = q
    dq = q - x
    part = jnp.sum(dq * dq, axis=(0, 1), keepdims=True)

    @pl.when(i == 0)
    def _():
        loss_ref[...] = jnp.zeros((1, 1), jnp.float32)

    loss_ref[...] += part

    @pl.when(i == n_steps - 1)
    def _():
        loss_ref[...] = loss_ref[...] / (
            jnp.float32(n_steps) * x.shape[0] * x.shape[1])


def _sc_gather(table128, idx128):
    info = plsc.get_sparse_core_info()
    n_workers = info.num_cores * info.num_subcores
    b_per_w = idx128.shape[0] // n_workers
    mesh = plsc.VectorSubcoreMesh(core_axis_name="c", subcore_axis_name="s")

    @functools.partial(
        pl.kernel, mesh=mesh,
        out_type=jax.ShapeDtypeStruct((idx128.shape[0], 128), jnp.float32),
        scratch_types=[
            pltpu.VMEM((b_per_w,), jnp.int32),
            pltpu.VMEM((b_per_w, 128), jnp.float32),
            pltpu.SemaphoreType.DMA,
        ],
    )
    def gather(table_hbm, idx_hbm, out_hbm, idx_v, rows_v, sem):
        wid = lax.axis_index("s") * info.num_cores + lax.axis_index("c")
        base = wid * b_per_w
        pltpu.sync_copy(idx_hbm.at[pl.ds(base, b_per_w)], idx_v)
        pltpu.async_copy(table_hbm.at[idx_v], rows_v, sem).wait()
        pltpu.sync_copy(rows_v, out_hbm.at[pl.ds(base, b_per_w)])

    return gather(table128, idx128)


@jax.jit
def kernel(inputs, embedding):
    n_tok = inputs.shape[0] * inputs.shape[1]
    flat = inputs.reshape(n_tok, _EMBED_DIM)
    grid = n_tok // _TILE

    e2, eb = pl.pallas_call(
        _prep_kernel,
        out_shape=[
            jax.ShapeDtypeStruct((1, _NUM_CODES), jnp.float32),
            jax.ShapeDtypeStruct((_NUM_CODES, _EMBED_DIM), jnp.bfloat16),
        ],
    )(embedding)

    idx3 = pl.pallas_call(
        _argmin_kernel,
        grid=(grid,),
        in_specs=[
            pl.BlockSpec((_TILE, _EMBED_DIM), lambda i: (i, 0)),
            pl.BlockSpec((_NUM_CODES, _EMBED_DIM), lambda i: (0, 0)),
            pl.BlockSpec((1, _NUM_CODES), lambda i: (0, 0)),
        ],
        out_specs=pl.BlockSpec((1, _TILE, 1), lambda i: (i, 0, 0)),
        out_shape=jax.ShapeDtypeStruct((grid, _TILE, 1), jnp.int32),
    )(flat, eb, e2)

    idx128 = idx3.reshape(n_tok) >> 2
    table128 = embedding.reshape(_NUM_CODES // _PACK, 128)
    q128 = _sc_gather(table128, idx128)

    ftile = 1024
    fgrid = n_tok // ftile
    qo, q, loss2 = pl.pallas_call(
        _finish_kernel,
        grid=(fgrid,),
        in_specs=[
            pl.BlockSpec((ftile, _EMBED_DIM), lambda i: (i, 0)),
            pl.BlockSpec((ftile, 128), lambda i: (i, 0)),
            pl.BlockSpec((ftile, 1), lambda i: (i, 0)),
        ],
        out_specs=[
            pl.BlockSpec((ftile, _EMBED_DIM), lambda i: (i, 0)),
            pl.BlockSpec((ftile, _EMBED_DIM), lambda i: (i, 0)),
            pl.BlockSpec((1, 1), lambda i: (0, 0)),
        ],
        out_shape=[
            jax.ShapeDtypeStruct((n_tok, _EMBED_DIM), jnp.float32),
            jax.ShapeDtypeStruct((n_tok, _EMBED_DIM), jnp.float32),
            jax.ShapeDtypeStruct((1, 1), jnp.float32),
        ],
    )(flat, q128, idx3.reshape(n_tok, 1))

    loss = loss2[0, 0]
    return (qo.reshape(inputs.shape), q.reshape(inputs.shape),
            loss, loss, idx3.reshape(inputs.shape[:-1]))


# speedup vs baseline: 2.5061x; 1.0014x over previous
import functools

import jax
import jax.numpy as jnp
from jax import lax
from jax.experimental import pallas as pl
from jax.experimental.pallas import tpu as pltpu
from jax.experimental.pallas import tpu_sc as plsc

_NUM_CODES = 8192
_EMBED_DIM = 32
_TILE = 256
_CHUNK = 2048
_LANES = 128
_PACK = 128 // _EMBED_DIM


def _prep_kernel(e_ref, e2_ref, eb_ref):
    e = e_ref[...]
    ones = jnp.ones((1, _EMBED_DIM), dtype=jnp.float32)
    e2_ref[...] = jax.lax.dot_general(
        ones, e * e,
        dimension_numbers=(((1,), (1,)), ((), ())),
        precision=jax.lax.Precision.HIGHEST,
        preferred_element_type=jnp.float32)
    eb_ref[...] = e.astype(jnp.bfloat16)


def _argmin_kernel(x_ref, eb_ref, e2_ref, idx_ref):
    x = x_ref[...]
    eb = eb_ref[...]

    xx = x * x
    r = xx[:, 0:8]
    for j in range(1, 4):
        r = r + xx[:, 8 * j:8 * j + 8]
    s = r[:, 0:4] + r[:, 4:8]
    s = s[:, 0:2] + s[:, 2:4]
    x2 = s[:, 0:1] + s[:, 1:2]

    xb = x.astype(jnp.bfloat16)
    e2 = e2_ref[...]
    lane = jax.lax.broadcasted_iota(jnp.int32, (_TILE, _LANES), 1)

    acc_v = None
    acc_i = None
    n_blk = _CHUNK // _LANES
    for c in range(_NUM_CODES // _CHUNK):
        mm = jax.lax.dot_general(
            xb, eb[c * _CHUNK:(c + 1) * _CHUNK, :],
            dimension_numbers=(((1,), (1,)), ((), ())),
            preferred_element_type=jnp.float32)
        run_v = None
        run_b = None
        for b in range(n_blk):
            lo = b * _LANES
            e2b = e2[:, c * _CHUNK + lo:c * _CHUNK + lo + _LANES]
            db = (x2 + e2b) - 2.0 * mm[:, lo:lo + _LANES]
            if run_v is None:
                run_v = db
                run_b = jnp.zeros((_TILE, _LANES), jnp.int32)
            else:
                upd = db < run_v
                run_v = jnp.where(upd, db, run_v)
                run_b = jnp.where(upd, b, run_b)
        md = jnp.min(run_v, axis=1, keepdims=True)
        cand = run_b * _LANES + lane
        ic = jnp.min(jnp.where(run_v == md, cand, _NUM_CODES),
                     axis=1, keepdims=True) + c * _CHUNK
        md_b = md.astype(jnp.bfloat16).astype(jnp.float32)
        if acc_v is None:
            acc_v, acc_i = md_b, ic
        else:
            upd = md < acc_v
            acc_v = jnp.where(upd, md_b, acc_v)
            acc_i = jnp.where(upd, ic, acc_i)

    idx_ref[...] = acc_i[None]


def _finish_kernel(x_ref, q128_ref, idx_ref, qo_ref, q_ref, loss_ref):
    i = pl.program_id(0)
    n_steps = pl.num_programs(0)
    x = x_ref[...]
    q128 = q128_ref[...]
    rem = idx_ref[0] & (_PACK - 1)

    q = jnp.where(rem == 0, q128[:, 0:_EMBED_DIM], 0.0)
    for k in range(1, _PACK):
        q = jnp.where(rem == k,
                      q128[:, k * _EMBED_DIM:(k + 1) * _EMBED_DIM], q)

    qo_ref[...] = x + (q - x)
    q_ref[...] = q
    dq = q - x
    part = jnp.sum(dq * dq, axis=(0, 1), keepdims=True)

    @pl.when(i == 0)
    def _():
        loss_ref[...] = jnp.zeros((1, 1), jnp.float32)

    loss_ref[...] += part

    @pl.when(i == n_steps - 1)
    def _():
        loss_ref[...] = loss_ref[...] / (
            jnp.float32(n_steps) * x.shape[0] * x.shape[1])


def _sc_gather(table128, idx128):
    info = plsc.get_sparse_core_info()
    n_workers = info.num_cores * info.num_subcores
    b_per_w = idx128.shape[0] // n_workers
    mesh = plsc.VectorSubcoreMesh(core_axis_name="c", subcore_axis_name="s")

    @functools.partial(
        pl.kernel, mesh=mesh,
        out_type=jax.ShapeDtypeStruct((idx128.shape[0], 128), jnp.float32),
        scratch_types=[
            pltpu.VMEM((b_per_w,), jnp.int32),
            pltpu.VMEM((b_per_w, 128), jnp.float32),
            pltpu.SemaphoreType.DMA,
        ],
    )
    def gather(table_hbm, idx_hbm, out_hbm, idx_v, rows_v, sem):
        wid = lax.axis_index("s") * info.num_cores + lax.axis_index("c")
        base = wid * b_per_w
        pltpu.sync_copy(idx_hbm.at[pl.ds(base, b_per_w)], idx_v)
        pltpu.async_copy(table_hbm.at[idx_v], rows_v, sem).wait()
        pltpu.sync_copy(rows_v, out_hbm.at[pl.ds(base, b_per_w)])

    return gather(table128, idx128)


@jax.jit
def kernel(inputs, embedding):
    n_tok = inputs.shape[0] * inputs.shape[1]
    flat = inputs.reshape(n_tok, _EMBED_DIM)
    grid = n_tok // _TILE

    e2, eb = pl.pallas_call(
        _prep_kernel,
        out_shape=[
            jax.ShapeDtypeStruct((1, _NUM_CODES), jnp.float32),
            jax.ShapeDtypeStruct((_NUM_CODES, _EMBED_DIM), jnp.bfloat16),
        ],
    )(embedding)

    idx3 = pl.pallas_call(
        _argmin_kernel,
        grid=(grid,),
        in_specs=[
            pl.BlockSpec((_TILE, _EMBED_DIM), lambda i: (i, 0)),
            pl.BlockSpec((_NUM_CODES, _EMBED_DIM), lambda i: (0, 0)),
            pl.BlockSpec((1, _NUM_CODES), lambda i: (0, 0)),
        ],
        out_specs=pl.BlockSpec((1, _TILE, 1), lambda i: (i, 0, 0)),
        out_shape=jax.ShapeDtypeStruct((grid, _TILE, 1), jnp.int32),
    )(flat, eb, e2)

    idx128 = idx3.reshape(n_tok) >> 2
    table128 = embedding.reshape(_NUM_CODES // _PACK, 128)
    q128 = _sc_gather(table128, idx128)

    ftile = 1024
    fgrid = n_tok // ftile
    qo, q, loss2 = pl.pallas_call(
        _finish_kernel,
        grid=(fgrid,),
        in_specs=[
            pl.BlockSpec((ftile, _EMBED_DIM), lambda i: (i, 0)),
            pl.BlockSpec((ftile, 128), lambda i: (i, 0)),
            pl.BlockSpec((1, ftile, 1), lambda i: (i, 0, 0)),
        ],
        out_specs=[
            pl.BlockSpec((ftile, _EMBED_DIM), lambda i: (i, 0)),
            pl.BlockSpec((ftile, _EMBED_DIM), lambda i: (i, 0)),
            pl.BlockSpec((1, 1), lambda i: (0, 0)),
        ],
        out_shape=[
            jax.ShapeDtypeStruct((n_tok, _EMBED_DIM), jnp.float32),
            jax.ShapeDtypeStruct((n_tok, _EMBED_DIM), jnp.float32),
            jax.ShapeDtypeStruct((1, 1), jnp.float32),
        ],
    )(flat, q128, idx3.reshape(fgrid, ftile, 1))

    loss = loss2[0, 0]
    return (qo.reshape(inputs.shape), q.reshape(inputs.shape),
            loss, loss, idx3.reshape(inputs.shape[:-1]))
